# baseline (speedup 1.0000x reference)
import jax
import jax.numpy as jnp
from jax.experimental import pallas as pl
from jax.experimental.pallas import tpu as pltpu

_TB = 1024
_CH = 512


def _w2sum_kernel(w2_ref, wg_ref, b1_ref, out_ref, wgt_ref, b1t_ref):
    E, Hn, _ = w2_ref.shape
    s = jnp.sum(w2_ref[...], axis=2)
    z = jnp.zeros((1, Hn), jnp.float32)
    row0 = jnp.concatenate([s[0:1, :], z], axis=1)
    row1 = jnp.concatenate([z, s[1:2, :]], axis=1)
    out_ref[...] = jnp.concatenate([row0, row1], axis=0)
    wgt_ref[...] = wg_ref[...].T
    b1t_ref[...] = b1_ref[...].T


def _moe_kernel(x_ref, wgt_ref, w1_ref, b1t_ref, b2_ref, w2sd_ref,
                out_ref, vals_ref):
    i = pl.program_id(0)
    Bn, Tn = out_ref.shape
    nblk = pl.num_programs(0)

    b2s = jnp.sum(b2_ref[...], axis=1, keepdims=True)
    w2sd = w2sd_ref[...]
    wgt = wgt_ref[...]
    dn = (((1,), (1,)), ((), ()))
    scale = 1.0 / (1.0 + 1e-9)

    dn_t = (((0,), (1,)), ((), ()))
    dn_r = (((1,), (0,)), ((), ()))
    for c in range(_TB // _CH):
        xc = x_ref[pl.ds(c * _CH, _CH), :]
        ht0 = jax.lax.dot_general(w1_ref[0], xc, dn_t,
                                  preferred_element_type=jnp.float32)
        ht1 = jax.lax.dot_general(w1_ref[1], xc, dn_t,
                                  preferred_element_type=jnp.float32)
        ht0 = jnp.maximum(ht0 + b1t_ref[:, 0:1], 0.0)
        ht1 = jnp.maximum(ht1 + b1t_ref[:, 1:2], 0.0)
        ht = jnp.concatenate([ht0, ht1], axis=0)
        s = jax.lax.dot_general(w2sd, ht, dn_r,
                                preferred_element_type=jnp.float32)
        s = s + b2s
        l = jax.lax.dot_general(wgt, xc, dn,
                                preferred_element_type=jnp.float32)
        c0 = jax.nn.sigmoid(l[0:1, :] - l[1:2, :]) * scale
        c1 = scale - c0
        val = c0 * s[0:1, :] + c1 * s[1:2, :]

        pos = i * _TB + c * _CH
        vals_ref[pl.ds(pos // Tn, 1), pl.ds(pos % Tn, _CH)] = val

    @pl.when(i == nblk - 1)
    def _():
        v = vals_ref[...]
        m = jnp.max(v, axis=1, keepdims=True)
        out_ref[...] = (v - m) - jnp.log(
            jnp.sum(jnp.exp(v - m), axis=1, keepdims=True))


def kernel(x, wg, W1, b1, W2, b2):
    B, T, D = x.shape
    E, _, H = W1.shape
    N = B * T
    nblk = N // _TB

    x2 = x.reshape(N, D)

    w2sd, wgT, b1T = pl.pallas_call(
        _w2sum_kernel,
        out_shape=(
            jax.ShapeDtypeStruct((E, 2 * H), jnp.float32),
            jax.ShapeDtypeStruct((E, D), jnp.float32),
            jax.ShapeDtypeStruct((H, E), jnp.float32),
        ),
    )(W2, wg, b1)

    out = pl.pallas_call(
        _moe_kernel,
        grid=(nblk,),
        in_specs=[
            pl.BlockSpec((_TB, D), lambda i: (i, 0)),
            pl.BlockSpec((E, D), lambda i: (0, 0)),
            pl.BlockSpec((E, D, H), lambda i: (0, 0, 0)),
            pl.BlockSpec((H, E), lambda i: (0, 0)),
            pl.BlockSpec((E, D), lambda i: (0, 0)),
            pl.BlockSpec((E, 2 * H), lambda i: (0, 0)),
        ],
        out_specs=pl.BlockSpec((B, T), lambda i: (0, 0)),
        out_shape=jax.ShapeDtypeStruct((B, T), jnp.float32),
        scratch_shapes=[
            pltpu.VMEM((B, T), jnp.float32),
        ],
        compiler_params=pltpu.CompilerParams(
            dimension_semantics=("arbitrary",),
        ),
    )(x2, wgT, W1, b1T, b2, w2sd)
    return out


# speedup vs baseline: 1.0194x; 1.0194x over previous
import jax
import jax.numpy as jnp
from jax.experimental import pallas as pl
from jax.experimental.pallas import tpu as pltpu

_TB = 1024
_CH = 512


def _w2sum_kernel(w2_ref, out_ref):
    E, Hn, _ = w2_ref.shape
    s = jnp.sum(w2_ref[...], axis=2)
    z = jnp.zeros((1, Hn), jnp.float32)
    row0 = jnp.concatenate([s[0:1, :], z], axis=1)
    row1 = jnp.concatenate([z, s[1:2, :]], axis=1)
    out_ref[...] = jnp.concatenate([row0, row1], axis=0)


def _moe_kernel(x_ref, wgt_ref, w1_ref, b1t_ref, b2_ref, w2sd_ref,
                out_ref, vals_ref):
    i = pl.program_id(0)
    Bn, Tn = out_ref.shape
    nblk = pl.num_programs(0)

    b2s = jnp.sum(b2_ref[...], axis=1, keepdims=True)
    w2sd = w2sd_ref[...]
    wgt = wgt_ref[...]
    dn = (((1,), (1,)), ((), ()))
    scale = 1.0 / (1.0 + 1e-9)

    dn_t = (((0,), (1,)), ((), ()))
    dn_r = (((1,), (0,)), ((), ()))
    for c in range(_TB // _CH):
        xc = x_ref[pl.ds(c * _CH, _CH), :]
        ht0 = jax.lax.dot_general(w1_ref[0], xc, dn_t,
                                  preferred_element_type=jnp.float32)
        ht1 = jax.lax.dot_general(w1_ref[1], xc, dn_t,
                                  preferred_element_type=jnp.float32)
        ht0 = jnp.maximum(ht0 + b1t_ref[:, 0:1], 0.0)
        ht1 = jnp.maximum(ht1 + b1t_ref[:, 1:2], 0.0)
        ht = jnp.concatenate([ht0, ht1], axis=0)
        s = jax.lax.dot_general(w2sd, ht, dn_r,
                                preferred_element_type=jnp.float32)
        s = s + b2s
        l = jax.lax.dot_general(wgt, xc, dn,
                                preferred_element_type=jnp.float32)
        c0 = jax.nn.sigmoid(l[0:1, :] - l[1:2, :]) * scale
        c1 = scale - c0
        val = c0 * s[0:1, :] + c1 * s[1:2, :]

        pos = i * _TB + c * _CH
        vals_ref[pl.ds(pos // Tn, 1), pl.ds(pos % Tn, _CH)] = val

    @pl.when(i == nblk - 1)
    def _():
        v = vals_ref[...]
        m = jnp.max(v, axis=1, keepdims=True)
        out_ref[...] = (v - m) - jnp.log(
            jnp.sum(jnp.exp(v - m), axis=1, keepdims=True))


def kernel(x, wg, W1, b1, W2, b2):
    B, T, D = x.shape
    E, _, H = W1.shape
    N = B * T
    nblk = N // _TB

    x2 = x.reshape(N, D)
    wgT = wg.T
    b1T = b1.T

    w2sd = pl.pallas_call(
        _w2sum_kernel,
        out_shape=jax.ShapeDtypeStruct((E, 2 * H), jnp.float32),
    )(W2)

    out = pl.pallas_call(
        _moe_kernel,
        grid=(nblk,),
        in_specs=[
            pl.BlockSpec((_TB, D), lambda i: (i, 0)),
            pl.BlockSpec((E, D), lambda i: (0, 0)),
            pl.BlockSpec((E, D, H), lambda i: (0, 0, 0)),
            pl.BlockSpec((H, E), lambda i: (0, 0)),
            pl.BlockSpec((E, D), lambda i: (0, 0)),
            pl.BlockSpec((E, 2 * H), lambda i: (0, 0)),
        ],
        out_specs=pl.BlockSpec((B, T), lambda i: (0, 0)),
        out_shape=jax.ShapeDtypeStruct((B, T), jnp.float32),
        scratch_shapes=[
            pltpu.VMEM((B, T), jnp.float32),
        ],
        compiler_params=pltpu.CompilerParams(
            dimension_semantics=("arbitrary",),
        ),
    )(x2, wgT, W1, b1T, b2, w2sd)
    return out
